# speedup vs baseline: 5.0324x; 5.0324x over previous
import functools

import jax
import jax.numpy as jnp
from jax import lax
from jax.experimental import pallas as pl
from jax.experimental.pallas import tpu as pltpu
from jax.experimental.pallas import tpu_sc as plsc

N, D, E = 10000, 128, 320000
NC, NS, L = 2, 16, 16
STRIPE = 640
ACC_ROWS = NS * STRIPE
CH = 80
EW = E // (NC * NS)
NCH = EW // CH


def _sc_aggregate(x, row, col):
    mesh = plsc.VectorSubcoreMesh(core_axis_name="c", subcore_axis_name="s")

    @functools.partial(
        pl.kernel,
        out_type=jax.ShapeDtypeStruct((NC, N, D), jnp.float32),
        mesh=mesh,
        scratch_types=[
            pltpu.VMEM_SHARED((ACC_ROWS, D), jnp.float32),
            pltpu.VMEM((CH,), jnp.int32),
            pltpu.VMEM((CH,), jnp.int32),
            pltpu.VMEM((CH,), jnp.int32),
            pltpu.VMEM((CH, D), jnp.float32),
            pltpu.VMEM((16, D), jnp.float32),
            pltpu.SemaphoreType.DMA,
        ],
    )
    def k(x_hbm, row_hbm, col_hbm, out_hbm, acc, row_v, col_v, dst_v,
          rows_v, buf_v, sem):
        c = lax.axis_index("c")
        s = lax.axis_index("s")
        wid = c * NS + s

        @pl.loop(0, 16)
        def _zb(i):
            @pl.loop(0, D, step=L)
            def _zl(j):
                buf_v[i, pl.ds(j, L)] = jnp.zeros((L,), jnp.float32)

        @pl.loop(0, STRIPE // 16)
        def _zs(i):
            pltpu.sync_copy(buf_v, acc.at[pl.ds(s * STRIPE + i * 16, 16)])

        plsc.subcore_barrier()

        @pl.loop(0, NCH)
        def _edges(g):
            base = wid * EW + g * CH
            pltpu.sync_copy(row_hbm.at[pl.ds(base, CH)], row_v)
            pltpu.sync_copy(col_hbm.at[pl.ds(base, CH)], col_v)

            @pl.loop(0, CH, step=L)
            def _dst(i):
                r = row_v[pl.ds(i, L)]
                cc = col_v[pl.ds(i, L)]
                dst_v[pl.ds(i, L)] = jnp.where(r == cc, N, r)

            pltpu.async_copy(x_hbm.at[col_v], rows_v, sem).wait()
            pltpu.sync_copy(rows_v, acc.at[dst_v], add=True)

        plsc.subcore_barrier()

        @pl.loop(0, STRIPE // 16)
        def _out(i):
            r0 = s * STRIPE + i * 16

            @pl.when(r0 + 16 <= N)
            def _():
                pltpu.sync_copy(acc.at[pl.ds(r0, 16)], buf_v)
                pltpu.sync_copy(buf_v, out_hbm.at[c, pl.ds(r0, 16)])

    return k(x, row, col)


def _mlp(x, a0, a1, W1, b1, W2, b2):
    BN = 1000

    def body(x_ref, a0_ref, a1_ref, W1_ref, b1_ref, W2_ref, b2_ref, o_ref):
        out = x_ref[...] + a0_ref[...] + a1_ref[...]
        h = lax.dot_general(out, W1_ref[...], (((1,), (0,)), ((), ())),
                            precision=lax.Precision.HIGHEST,
                            preferred_element_type=jnp.float32)
        h = jnp.maximum(h + b1_ref[...], 0.0)
        y = lax.dot_general(h, W2_ref[...], (((1,), (0,)), ((), ())),
                            precision=lax.Precision.HIGHEST,
                            preferred_element_type=jnp.float32)
        o_ref[...] = y + b2_ref[...]

    nd = pl.BlockSpec((BN, D), lambda i: (i, 0))
    w = pl.BlockSpec((D, D), lambda i: (0, 0))
    b = pl.BlockSpec((1, D), lambda i: (0, 0))
    return pl.pallas_call(
        body,
        grid=(N // BN,),
        in_specs=[nd, nd, nd, w, b, w, b],
        out_specs=nd,
        out_shape=jax.ShapeDtypeStruct((N, D), jnp.float32),
    )(x, a0, a1, W1, b1, W2, b2)


def kernel(x, edge_index, W1, b1, W2, b2):
    row = edge_index[0]
    col = edge_index[1]
    agg = _sc_aggregate(x, row, col)
    return _mlp(x, agg[0], agg[1], W1, b1.reshape(1, D), W2, b2.reshape(1, D))


# speedup vs baseline: 11.0317x; 2.1921x over previous
import functools

import jax
import jax.numpy as jnp
from jax import lax
from jax.experimental import pallas as pl
from jax.experimental.pallas import tpu as pltpu
from jax.experimental.pallas import tpu_sc as plsc

N, D, E = 10000, 128, 320000
NC, NS, L = 2, 16, 16
NW = NC * NS
STRIPE = 640
ACC_ROWS = NS * STRIPE
CH = 80
EW = E // NW
NCH = EW // CH
NB = 4


def _dst_prep(row2, col2):
    def body(r_ref, c_ref, o_ref):
        r = r_ref[...]
        o_ref[...] = jnp.where(r == c_ref[...], N, r)

    bs = pl.BlockSpec((E // 128, 128), lambda i: (i, 0))
    return pl.pallas_call(
        body,
        grid=(1,),
        in_specs=[bs, bs],
        out_specs=bs,
        out_shape=jax.ShapeDtypeStruct((E // 128, 128), jnp.int32),
    )(row2, col2)


def _sc_aggregate(x, col, dst):
    mesh = plsc.VectorSubcoreMesh(core_axis_name="c", subcore_axis_name="s")

    @functools.partial(
        pl.kernel,
        out_type=jax.ShapeDtypeStruct((NC, N, D), jnp.float32),
        mesh=mesh,
        scratch_types=[
            pltpu.VMEM_SHARED((ACC_ROWS, D), jnp.float32),
            pltpu.VMEM((NB, CH), jnp.int32),
            pltpu.VMEM((NB, CH), jnp.int32),
            pltpu.VMEM((NB, CH, D), jnp.float32),
            pltpu.VMEM((16, D), jnp.float32),
            pltpu.SemaphoreType.DMA,
            pltpu.SemaphoreType.DMA,
            pltpu.SemaphoreType.DMA,
        ],
    )
    def k(x_hbm, col_hbm, dst_hbm, out_hbm, acc, col_v, dst_v, rows_v,
          buf_v, sem_i, sem_g, sem_z):
        c = lax.axis_index("c")
        s = lax.axis_index("s")
        wid = c * NS + s
        ebase = wid * EW

        @pl.loop(0, 16)
        def _zb(i):
            @pl.loop(0, D, step=L)
            def _zl(j):
                buf_v[i, pl.ds(j, L)] = jnp.zeros((L,), jnp.float32)

        @pl.loop(0, STRIPE // 16)
        def _zs(i):
            pltpu.async_copy(buf_v, acc.at[pl.ds(s * STRIPE + i * 16, 16)],
                             sem_z)

        @pl.loop(0, STRIPE // 16)
        def _zw(i):
            pltpu.make_async_copy(buf_v, acc.at[pl.ds(0, 16)], sem_z).wait()

        plsc.subcore_barrier()

        def issue_idx(g, b):
            pltpu.async_copy(col_hbm.at[pl.ds(ebase + g * CH, CH)],
                             col_v.at[b], sem_i)
            pltpu.async_copy(dst_hbm.at[pl.ds(ebase + g * CH, CH)],
                             dst_v.at[b], sem_i)

        def wait_idx_issue_gather(g, b):
            pltpu.make_async_copy(col_hbm.at[pl.ds(0, CH)], col_v.at[0],
                                  sem_i).wait()
            pltpu.make_async_copy(col_hbm.at[pl.ds(0, CH)], col_v.at[0],
                                  sem_i).wait()
            pltpu.async_copy(x_hbm.at[col_v.at[b]], rows_v.at[b], sem_g)

        for p in range(NB - 1):
            issue_idx(p, p)
        for p in range(NB - 2):
            wait_idx_issue_gather(p, p)

        @pl.loop(0, NCH)
        def _edges(g):
            b = lax.rem(g, NB)

            @pl.when(g + (NB - 1) < NCH)
            def _():
                issue_idx(g + NB - 1, lax.rem(g + NB - 1, NB))

            @pl.when(g + (NB - 2) < NCH)
            def _():
                wait_idx_issue_gather(g + NB - 2, lax.rem(g + NB - 2, NB))

            pltpu.make_async_copy(x_hbm.at[col_v.at[b]], rows_v.at[b],
                                  sem_g).wait()
            pltpu.sync_copy(rows_v.at[b], acc.at[dst_v.at[b]], add=True)

        plsc.subcore_barrier()

        @pl.when(s < NS - 1)
        def _full():
            pltpu.sync_copy(acc.at[pl.ds(s * STRIPE, STRIPE)],
                            out_hbm.at[c, pl.ds(s * STRIPE, STRIPE)])

        @pl.when(s == NS - 1)
        def _tail():
            r0 = (NS - 1) * STRIPE
            pltpu.sync_copy(acc.at[pl.ds(r0, N - r0)],
                            out_hbm.at[c, pl.ds(r0, N - r0)])

    return k(x, col, dst)


def _mlp(x, a0, a1, W1, b1, W2, b2):
    BN = 1000

    def body(x_ref, a0_ref, a1_ref, W1_ref, b1_ref, W2_ref, b2_ref, o_ref):
        out = x_ref[...] + a0_ref[...] + a1_ref[...]
        h = lax.dot_general(out, W1_ref[...], (((1,), (0,)), ((), ())),
                            precision=lax.Precision.HIGHEST,
                            preferred_element_type=jnp.float32)
        h = jnp.maximum(h + b1_ref[...], 0.0)
        y = lax.dot_general(h, W2_ref[...], (((1,), (0,)), ((), ())),
                            precision=lax.Precision.HIGHEST,
                            preferred_element_type=jnp.float32)
        o_ref[...] = y + b2_ref[...]

    nd = pl.BlockSpec((BN, D), lambda i: (i, 0))
    w = pl.BlockSpec((D, D), lambda i: (0, 0))
    b = pl.BlockSpec((1, D), lambda i: (0, 0))
    return pl.pallas_call(
        body,
        grid=(N // BN,),
        in_specs=[nd, nd, nd, w, b, w, b],
        out_specs=nd,
        out_shape=jax.ShapeDtypeStruct((N, D), jnp.float32),
    )(x, a0, a1, W1, b1, W2, b2)


def kernel(x, edge_index, W1, b1, W2, b2):
    row = edge_index[0]
    col = edge_index[1]
    dst = _dst_prep(row.reshape(E // 128, 128),
                    col.reshape(E // 128, 128)).reshape(E)
    agg = _sc_aggregate(x, col, dst)
    return _mlp(x, agg[0], agg[1], W1, b1.reshape(1, D), W2, b2.reshape(1, D))


# speedup vs baseline: 11.2635x; 1.0210x over previous
import functools

import jax
import jax.numpy as jnp
from jax import lax
from jax.experimental import pallas as pl
from jax.experimental.pallas import tpu as pltpu
from jax.experimental.pallas import tpu_sc as plsc

N, D, E = 10000, 128, 320000
NC, NS, L = 2, 16, 16
NW = NC * NS
STRIPE = 640
ACC_ROWS = NS * STRIPE
CH = 80
EW = E // NW
NCH = EW // CH
NB = 4
NSUP = (NCH + NB - 1) // NB


def _sc_aggregate(x, row, col):
    mesh = plsc.VectorSubcoreMesh(core_axis_name="c", subcore_axis_name="s")

    @functools.partial(
        pl.kernel,
        out_type=jax.ShapeDtypeStruct((NC, N, D), jnp.float32),
        mesh=mesh,
        scratch_types=[
            pltpu.VMEM_SHARED((ACC_ROWS, D), jnp.float32),
            [pltpu.VMEM((CH,), jnp.int32) for _ in range(NB)],
            [pltpu.VMEM((CH,), jnp.int32) for _ in range(NB)],
            [pltpu.VMEM((CH,), jnp.int32) for _ in range(NB)],
            [pltpu.VMEM((CH, D), jnp.float32) for _ in range(NB)],
            pltpu.VMEM((16, D), jnp.float32),
            pltpu.SemaphoreType.DMA,
            pltpu.SemaphoreType.DMA,
            pltpu.SemaphoreType.DMA,
        ],
    )
    def k(x_hbm, row_hbm, col_hbm, out_hbm, acc, row_r, col_r, dst_r,
          rows_r, buf_v, sem_i, sem_g, sem_z):
        c = lax.axis_index("c")
        s = lax.axis_index("s")
        ebase = (c * NS + s) * EW

        @pl.loop(0, 16)
        def _zb(i):
            @pl.loop(0, D, step=L)
            def _zl(j):
                buf_v[i, pl.ds(j, L)] = jnp.zeros((L,), jnp.float32)

        @pl.loop(0, STRIPE // 16)
        def _zs(i):
            pltpu.async_copy(buf_v, acc.at[pl.ds(s * STRIPE + i * 16, 16)],
                             sem_z)

        @pl.loop(0, STRIPE // 16)
        def _zw(i):
            pltpu.make_async_copy(buf_v, acc.at[pl.ds(0, 16)], sem_z).wait()

        plsc.subcore_barrier()

        def issue_idx(g, b):
            pltpu.async_copy(row_hbm.at[pl.ds(ebase + g * CH, CH)],
                             row_r[b], sem_i)
            pltpu.async_copy(col_hbm.at[pl.ds(ebase + g * CH, CH)],
                             col_r[b], sem_i)

        def prep_gather(g, b):
            pltpu.make_async_copy(row_hbm.at[pl.ds(0, CH)], row_r[0],
                                  sem_i).wait()
            pltpu.make_async_copy(row_hbm.at[pl.ds(0, CH)], row_r[0],
                                  sem_i).wait()

            @pl.loop(0, CH, step=L)
            def _dst(i):
                r = row_r[b][pl.ds(i, L)]
                cc = col_r[b][pl.ds(i, L)]
                dst_r[b][pl.ds(i, L)] = jnp.where(r == cc, N, r)

            pltpu.async_copy(x_hbm.at[col_r[b]], rows_r[b], sem_g)

        def retire(g, b):
            pltpu.make_async_copy(x_hbm.at[col_r[b]], rows_r[b],
                                  sem_g).wait()
            pltpu.sync_copy(rows_r[b], acc.at[dst_r[b]], add=True)

        for p in range(NB - 1):
            issue_idx(p, p)
        for p in range(NB - 2):
            prep_gather(p, p)

        @pl.loop(0, NSUP)
        def _edges(sup):
            g0 = sup * NB
            for slot in range(NB):
                g = g0 + slot

                @pl.when(g + (NB - 1) < NCH)
                def _():
                    issue_idx(g + NB - 1, (slot + NB - 1) % NB)

                @pl.when(g + (NB - 2) < NCH)
                def _():
                    prep_gather(g + NB - 2, (slot + NB - 2) % NB)

                @pl.when(g < NCH)
                def _():
                    retire(g, slot)

        plsc.subcore_barrier()

        @pl.when(s < NS - 1)
        def _full():
            pltpu.sync_copy(acc.at[pl.ds(s * STRIPE, STRIPE)],
                            out_hbm.at[c, pl.ds(s * STRIPE, STRIPE)])

        @pl.when(s == NS - 1)
        def _tail():
            r0 = (NS - 1) * STRIPE
            pltpu.sync_copy(acc.at[pl.ds(r0, N - r0)],
                            out_hbm.at[c, pl.ds(r0, N - r0)])

    return k(x, row, col)


def _mlp(x, a0, a1, W1, b1, W2, b2):
    BN = 1000

    def body(x_ref, a0_ref, a1_ref, W1_ref, b1_ref, W2_ref, b2_ref, o_ref):
        out = x_ref[...] + a0_ref[...] + a1_ref[...]
        h = lax.dot_general(out, W1_ref[...], (((1,), (0,)), ((), ())),
                            precision=lax.Precision.HIGHEST,
                            preferred_element_type=jnp.float32)
        h = jnp.maximum(h + b1_ref[...], 0.0)
        y = lax.dot_general(h, W2_ref[...], (((1,), (0,)), ((), ())),
                            precision=lax.Precision.HIGHEST,
                            preferred_element_type=jnp.float32)
        o_ref[...] = y + b2_ref[...]

    nd = pl.BlockSpec((BN, D), lambda i: (i, 0))
    w = pl.BlockSpec((D, D), lambda i: (0, 0))
    b = pl.BlockSpec((1, D), lambda i: (0, 0))
    return pl.pallas_call(
        body,
        grid=(N // BN,),
        in_specs=[nd, nd, nd, w, b, w, b],
        out_specs=nd,
        out_shape=jax.ShapeDtypeStruct((N, D), jnp.float32),
    )(x, a0, a1, W1, b1, W2, b2)


def kernel(x, edge_index, W1, b1, W2, b2):
    agg = _sc_aggregate(x, edge_index[0], edge_index[1])
    return _mlp(x, agg[0], agg[1], W1, b1.reshape(1, D), W2, b2.reshape(1, D))


# speedup vs baseline: 11.7422x; 1.0425x over previous
import functools

import jax
import jax.numpy as jnp
from jax import lax
from jax.experimental import pallas as pl
from jax.experimental.pallas import tpu as pltpu
from jax.experimental.pallas import tpu_sc as plsc

N, D, E = 10000, 128, 320000
NC, NS, L = 2, 16, 16
NW = NC * NS
STRIPE = 640
ACC_ROWS = NS * STRIPE
CH = 80
EW = E // NW
NCH = EW // CH
NB = 4
NSUP = (NCH + NB - 1) // NB


def _sc_aggregate(x, row, col):
    mesh = plsc.VectorSubcoreMesh(core_axis_name="c", subcore_axis_name="s")

    @functools.partial(
        pl.kernel,
        out_type=jax.ShapeDtypeStruct((NC, N, D), jnp.float32),
        mesh=mesh,
        scratch_types=[
            pltpu.VMEM_SHARED((ACC_ROWS, D), jnp.float32),
            [pltpu.VMEM((CH,), jnp.int32) for _ in range(NB)],
            [pltpu.VMEM((CH,), jnp.int32) for _ in range(NB)],
            [pltpu.VMEM((CH,), jnp.int32) for _ in range(NB)],
            [pltpu.VMEM((CH, D), jnp.float32) for _ in range(NB)],
            pltpu.VMEM((16, D), jnp.float32),
            pltpu.SemaphoreType.DMA,
            pltpu.SemaphoreType.DMA,
            pltpu.SemaphoreType.DMA,
            pltpu.SemaphoreType.DMA,
        ],
    )
    def k(x_hbm, row_hbm, col_hbm, out_hbm, acc, row_r, col_r, dst_r,
          rows_r, buf_v, sem_i, sem_g, sem_s, sem_z):
        c = lax.axis_index("c")
        s = lax.axis_index("s")
        ebase = (c * NS + s) * EW

        @pl.loop(0, 16)
        def _zb(i):
            @pl.loop(0, D, step=L)
            def _zl(j):
                buf_v[i, pl.ds(j, L)] = jnp.zeros((L,), jnp.float32)

        @pl.loop(0, STRIPE // 16)
        def _zs(i):
            pltpu.async_copy(buf_v, acc.at[pl.ds(s * STRIPE + i * 16, 16)],
                             sem_z)

        @pl.loop(0, STRIPE // 16)
        def _zw(i):
            pltpu.make_async_copy(buf_v, acc.at[pl.ds(0, 16)], sem_z).wait()

        plsc.subcore_barrier()

        def issue_idx(g, b):
            pltpu.async_copy(row_hbm.at[pl.ds(ebase + g * CH, CH)],
                             row_r[b], sem_i)
            pltpu.async_copy(col_hbm.at[pl.ds(ebase + g * CH, CH)],
                             col_r[b], sem_i)

        def prep_gather(g, b):
            pltpu.make_async_copy(row_hbm.at[pl.ds(0, CH)], row_r[0],
                                  sem_i).wait()
            pltpu.make_async_copy(row_hbm.at[pl.ds(0, CH)], row_r[0],
                                  sem_i).wait()

            @pl.loop(0, CH, step=L)
            def _dst(i):
                r = row_r[b][pl.ds(i, L)]
                cc = col_r[b][pl.ds(i, L)]
                dst_r[b][pl.ds(i, L)] = jnp.where(r == cc, N, r)

            pltpu.async_copy(x_hbm.at[col_r[b]], rows_r[b], sem_g)

        def retire(g, b):
            pltpu.make_async_copy(x_hbm.at[col_r[b]], rows_r[b],
                                  sem_g).wait()
            pltpu.async_copy(rows_r[b], acc.at[dst_r[b]], sem_s, add=True)

        def drain_scatter():
            pltpu.make_async_copy(rows_r[0], acc.at[dst_r[0]], sem_s).wait()

        for p in range(NB - 1):
            issue_idx(p, p)
        for p in range(NB - 2):
            prep_gather(p, p)

        @pl.loop(0, NSUP)
        def _edges(sup):
            g0 = sup * NB
            for slot in range(NB):
                g = g0 + slot

                @pl.when(g + (NB - 1) < NCH)
                def _():
                    issue_idx(g + NB - 1, (slot + NB - 1) % NB)

                @pl.when((g >= 2) & (g - 2 < NCH))
                def _():
                    drain_scatter()

                @pl.when(g + (NB - 2) < NCH)
                def _():
                    prep_gather(g + NB - 2, (slot + NB - 2) % NB)

                @pl.when(g < NCH)
                def _():
                    retire(g, slot)

        plsc.subcore_barrier()

        @pl.when(s < NS - 1)
        def _full():
            pltpu.sync_copy(acc.at[pl.ds(s * STRIPE, STRIPE)],
                            out_hbm.at[c, pl.ds(s * STRIPE, STRIPE)])

        @pl.when(s == NS - 1)
        def _tail():
            r0 = (NS - 1) * STRIPE
            pltpu.sync_copy(acc.at[pl.ds(r0, N - r0)],
                            out_hbm.at[c, pl.ds(r0, N - r0)])

    return k(x, row, col)


def _mlp(x, a0, a1, W1, b1, W2, b2):
    BN = 1000

    def body(x_ref, a0_ref, a1_ref, W1_ref, b1_ref, W2_ref, b2_ref, o_ref):
        out = x_ref[...] + a0_ref[...] + a1_ref[...]
        h = lax.dot_general(out, W1_ref[...], (((1,), (0,)), ((), ())),
                            precision=lax.Precision.HIGHEST,
                            preferred_element_type=jnp.float32)
        h = jnp.maximum(h + b1_ref[...], 0.0)
        y = lax.dot_general(h, W2_ref[...], (((1,), (0,)), ((), ())),
                            precision=lax.Precision.HIGHEST,
                            preferred_element_type=jnp.float32)
        o_ref[...] = y + b2_ref[...]

    nd = pl.BlockSpec((BN, D), lambda i: (i, 0))
    w = pl.BlockSpec((D, D), lambda i: (0, 0))
    b = pl.BlockSpec((1, D), lambda i: (0, 0))
    return pl.pallas_call(
        body,
        grid=(N // BN,),
        in_specs=[nd, nd, nd, w, b, w, b],
        out_specs=nd,
        out_shape=jax.ShapeDtypeStruct((N, D), jnp.float32),
    )(x, a0, a1, W1, b1, W2, b2)


def kernel(x, edge_index, W1, b1, W2, b2):
    agg = _sc_aggregate(x, edge_index[0], edge_index[1])
    return _mlp(x, agg[0], agg[1], W1, b1.reshape(1, D), W2, b2.reshape(1, D))


# speedup vs baseline: 14.2073x; 1.2099x over previous
import functools

import jax
import jax.numpy as jnp
from jax import lax
from jax.experimental import pallas as pl
from jax.experimental.pallas import tpu as pltpu
from jax.experimental.pallas import tpu_sc as plsc

N, D, E = 10000, 128, 320000
NC, NS, L = 2, 16, 16
NW = NC * NS
STRIPE = 640
ACC_ROWS = NS * STRIPE
CH = 80
EW = E // NW
NCH = EW // CH
NB = 4
NSUP = (NCH + NB - 1) // NB


def _sc_aggregate(x, row, col):
    mesh = plsc.VectorSubcoreMesh(core_axis_name="c", subcore_axis_name="s")

    @functools.partial(
        pl.kernel,
        out_type=jax.ShapeDtypeStruct((NC, N, D), jnp.float32),
        mesh=mesh,
        scratch_types=[
            pltpu.VMEM_SHARED((ACC_ROWS, D), jnp.float32),
            [pltpu.VMEM((2, CH), jnp.int32) for _ in range(NB)],
            [pltpu.VMEM((CH,), jnp.int32) for _ in range(NB)],
            [pltpu.VMEM((CH, D), jnp.float32) for _ in range(NB)],
            pltpu.VMEM((16, D), jnp.float32),
            pltpu.SemaphoreType.DMA,
            pltpu.SemaphoreType.DMA,
            pltpu.SemaphoreType.DMA,
            pltpu.SemaphoreType.DMA,
        ],
    )
    def k(x_hbm, row_hbm, col_hbm, out_hbm, acc, ei_r, dst_r,
          rows_r, buf_v, sem_i, sem_g, sem_s, sem_z):
        c = lax.axis_index("c")
        s = lax.axis_index("s")
        ebase = (c * NS + s) * EW

        @pl.loop(0, 16)
        def _zb(i):
            @pl.loop(0, D, step=L)
            def _zl(j):
                buf_v[i, pl.ds(j, L)] = jnp.zeros((L,), jnp.float32)

        @pl.loop(0, STRIPE // 16)
        def _zs(i):
            pltpu.async_copy(buf_v, acc.at[pl.ds(s * STRIPE + i * 16, 16)],
                             sem_z)

        @pl.loop(0, STRIPE // 16)
        def _zw(i):
            pltpu.make_async_copy(buf_v, acc.at[pl.ds(0, 16)], sem_z).wait()

        plsc.subcore_barrier()

        def issue_idx(g, b):
            pltpu.async_copy(row_hbm.at[pl.ds(ebase + g * CH, CH)],
                             ei_r[b].at[0], sem_i)
            pltpu.async_copy(col_hbm.at[pl.ds(ebase + g * CH, CH)],
                             ei_r[b].at[1], sem_i)

        def prep_gather(g, b):
            pltpu.make_async_copy(row_hbm.at[pl.ds(0, CH)], ei_r[0].at[0],
                                  sem_i).wait()
            pltpu.make_async_copy(row_hbm.at[pl.ds(0, CH)], ei_r[0].at[0],
                                  sem_i).wait()

            @pl.loop(0, CH, step=L)
            def _dst(i):
                r = ei_r[b][0, pl.ds(i, L)]
                cc = ei_r[b][1, pl.ds(i, L)]
                dst_r[b][pl.ds(i, L)] = jnp.where(r == cc, N, r)

            pltpu.async_copy(x_hbm.at[ei_r[b].at[1]], rows_r[b], sem_g)

        def retire(g, b):
            pltpu.make_async_copy(x_hbm.at[ei_r[b].at[1]], rows_r[b],
                                  sem_g).wait()
            pltpu.async_copy(rows_r[b], acc.at[dst_r[b]], sem_s, add=True)

        def drain_scatter():
            pltpu.make_async_copy(rows_r[0], acc.at[dst_r[0]], sem_s).wait()

        for p in range(NB - 1):
            issue_idx(p, p)
        for p in range(NB - 2):
            prep_gather(p, p)

        @pl.loop(0, NSUP)
        def _edges(sup):
            g0 = sup * NB
            for slot in range(NB):
                g = g0 + slot

                @pl.when(g + (NB - 1) < NCH)
                def _():
                    issue_idx(g + NB - 1, (slot + NB - 1) % NB)

                @pl.when((g >= 2) & (g - 2 < NCH))
                def _():
                    drain_scatter()

                @pl.when(g + (NB - 2) < NCH)
                def _():
                    prep_gather(g + NB - 2, (slot + NB - 2) % NB)

                @pl.when(g < NCH)
                def _():
                    retire(g, slot)

        plsc.subcore_barrier()

        @pl.when(s < NS - 1)
        def _full():
            pltpu.sync_copy(acc.at[pl.ds(s * STRIPE, STRIPE)],
                            out_hbm.at[c, pl.ds(s * STRIPE, STRIPE)])

        @pl.when(s == NS - 1)
        def _tail():
            r0 = (NS - 1) * STRIPE
            pltpu.sync_copy(acc.at[pl.ds(r0, N - r0)],
                            out_hbm.at[c, pl.ds(r0, N - r0)])

    return k(x, row, col)


def _mlp(x, agg, W1, b1, W2, b2):
    BN = 1000

    def body(x_ref, a0_ref, a1_ref, W1_ref, b1_ref, W2_ref, b2_ref, o_ref):
        out = x_ref[...] + a0_ref[0] + a1_ref[0]
        h = lax.dot_general(out, W1_ref[...], (((1,), (0,)), ((), ())),
                            precision=lax.Precision.DEFAULT,
                            preferred_element_type=jnp.float32)
        h = jnp.maximum(h + b1_ref[...], 0.0)
        y = lax.dot_general(h, W2_ref[...], (((1,), (0,)), ((), ())),
                            precision=lax.Precision.DEFAULT,
                            preferred_element_type=jnp.float32)
        o_ref[...] = y + b2_ref[...]

    nd = pl.BlockSpec((BN, D), lambda i: (i, 0))
    a0 = pl.BlockSpec((1, BN, D), lambda i: (0, i, 0))
    a1 = pl.BlockSpec((1, BN, D), lambda i: (1, i, 0))
    w = pl.BlockSpec((D, D), lambda i: (0, 0))
    b = pl.BlockSpec((1, D), lambda i: (0, 0))
    return pl.pallas_call(
        body,
        grid=(N // BN,),
        in_specs=[nd, a0, a1, w, b, w, b],
        out_specs=nd,
        out_shape=jax.ShapeDtypeStruct((N, D), jnp.float32),
    )(x, agg, agg, W1, b1, W2, b2)


def kernel(x, edge_index, W1, b1, W2, b2):
    agg = _sc_aggregate(x, edge_index[0], edge_index[1])
    return _mlp(x, agg, W1, b1.reshape(1, D), W2, b2.reshape(1, D))


# speedup vs baseline: 14.5506x; 1.0242x over previous
import functools

import jax
import jax.numpy as jnp
from jax import lax
from jax.experimental import pallas as pl
from jax.experimental.pallas import tpu as pltpu
from jax.experimental.pallas import tpu_sc as plsc

N, D, E = 10000, 128, 320000
NC, NS, L = 2, 16, 16
NW = NC * NS
STRIPE = 640
ACC_ROWS = NS * STRIPE
CH = 40
EW = E // NW
NCH = EW // CH
NB = 8
IDX_D = 7
G_D = 5
SC_LAG = NB - G_D
NSUP = (NCH + NB - 1) // NB


def _sc_aggregate(x, row, col):
    mesh = plsc.VectorSubcoreMesh(core_axis_name="c", subcore_axis_name="s")

    @functools.partial(
        pl.kernel,
        out_type=jax.ShapeDtypeStruct((NC, N, D), jnp.float32),
        mesh=mesh,
        scratch_types=[
            pltpu.VMEM_SHARED((ACC_ROWS, D), jnp.float32),
            [pltpu.VMEM((2, CH), jnp.int32) for _ in range(NB)],
            [pltpu.VMEM((CH,), jnp.int32) for _ in range(NB)],
            [pltpu.VMEM((CH, D), jnp.float32) for _ in range(NB)],
            pltpu.VMEM((16, D), jnp.float32),
            pltpu.SemaphoreType.DMA,
            pltpu.SemaphoreType.DMA,
            pltpu.SemaphoreType.DMA,
            pltpu.SemaphoreType.DMA,
        ],
    )
    def k(x_hbm, row_hbm, col_hbm, out_hbm, acc, ei_r, dst_r,
          rows_r, buf_v, sem_i, sem_g, sem_s, sem_z):
        c = lax.axis_index("c")
        s = lax.axis_index("s")
        ebase = (c * NS + s) * EW

        @pl.loop(0, 16)
        def _zb(i):
            @pl.loop(0, D, step=L)
            def _zl(j):
                buf_v[i, pl.ds(j, L)] = jnp.zeros((L,), jnp.float32)

        @pl.loop(0, STRIPE // 16)
        def _zs(i):
            pltpu.async_copy(buf_v, acc.at[pl.ds(s * STRIPE + i * 16, 16)],
                             sem_z)

        @pl.loop(0, STRIPE // 16)
        def _zw(i):
            pltpu.make_async_copy(buf_v, acc.at[pl.ds(0, 16)], sem_z).wait()

        plsc.subcore_barrier()

        def issue_idx(g, b):
            pltpu.async_copy(row_hbm.at[pl.ds(ebase + g * CH, CH)],
                             ei_r[b].at[0], sem_i)
            pltpu.async_copy(col_hbm.at[pl.ds(ebase + g * CH, CH)],
                             ei_r[b].at[1], sem_i)

        def prep_gather(g, b):
            pltpu.make_async_copy(row_hbm.at[pl.ds(0, CH)], ei_r[0].at[0],
                                  sem_i).wait()
            pltpu.make_async_copy(row_hbm.at[pl.ds(0, CH)], ei_r[0].at[0],
                                  sem_i).wait()

            @pl.loop(0, CH, step=L)
            def _dst(i):
                r = ei_r[b][0, pl.ds(i, L)]
                cc = ei_r[b][1, pl.ds(i, L)]
                dst_r[b][pl.ds(i, L)] = jnp.where(r == cc, N, r)

            pltpu.async_copy(x_hbm.at[ei_r[b].at[1]], rows_r[b], sem_g)

        def retire(g, b):
            pltpu.make_async_copy(x_hbm.at[ei_r[b].at[1]], rows_r[b],
                                  sem_g).wait()
            pltpu.async_copy(rows_r[b], acc.at[dst_r[b]], sem_s, add=True)

        def drain_scatter():
            pltpu.make_async_copy(rows_r[0], acc.at[dst_r[0]], sem_s).wait()

        for p in range(IDX_D):
            issue_idx(p, p)
        for p in range(G_D):
            prep_gather(p, p)

        @pl.loop(0, NSUP)
        def _edges(sup):
            g0 = sup * NB
            for slot in range(NB):
                g = g0 + slot

                @pl.when(g + IDX_D < NCH)
                def _():
                    issue_idx(g + IDX_D, (slot + IDX_D) % NB)

                @pl.when((g >= SC_LAG) & (g - SC_LAG < NCH))
                def _():
                    drain_scatter()

                @pl.when(g + G_D < NCH)
                def _():
                    prep_gather(g + G_D, (slot + G_D) % NB)

                @pl.when(g < NCH)
                def _():
                    retire(g, slot)

        plsc.subcore_barrier()

        @pl.when(s < NS - 1)
        def _full():
            pltpu.sync_copy(acc.at[pl.ds(s * STRIPE, STRIPE)],
                            out_hbm.at[c, pl.ds(s * STRIPE, STRIPE)])

        @pl.when(s == NS - 1)
        def _tail():
            r0 = (NS - 1) * STRIPE
            pltpu.sync_copy(acc.at[pl.ds(r0, N - r0)],
                            out_hbm.at[c, pl.ds(r0, N - r0)])

    return k(x, row, col)


def _mlp(x, agg, W1, b1, W2, b2):
    BN = 1000

    def body(x_ref, a0_ref, a1_ref, W1_ref, b1_ref, W2_ref, b2_ref, o_ref):
        out = x_ref[...] + a0_ref[0] + a1_ref[0]
        h = lax.dot_general(out, W1_ref[...], (((1,), (0,)), ((), ())),
                            precision=lax.Precision.DEFAULT,
                            preferred_element_type=jnp.float32)
        h = jnp.maximum(h + b1_ref[...], 0.0)
        y = lax.dot_general(h, W2_ref[...], (((1,), (0,)), ((), ())),
                            precision=lax.Precision.DEFAULT,
                            preferred_element_type=jnp.float32)
        o_ref[...] = y + b2_ref[...]

    nd = pl.BlockSpec((BN, D), lambda i: (i, 0))
    a0 = pl.BlockSpec((1, BN, D), lambda i: (0, i, 0))
    a1 = pl.BlockSpec((1, BN, D), lambda i: (1, i, 0))
    w = pl.BlockSpec((D, D), lambda i: (0, 0))
    b = pl.BlockSpec((1, D), lambda i: (0, 0))
    return pl.pallas_call(
        body,
        grid=(N // BN,),
        in_specs=[nd, a0, a1, w, b, w, b],
        out_specs=nd,
        out_shape=jax.ShapeDtypeStruct((N, D), jnp.float32),
    )(x, agg, agg, W1, b1, W2, b2)


def kernel(x, edge_index, W1, b1, W2, b2):
    agg = _sc_aggregate(x, edge_index[0], edge_index[1])
    return _mlp(x, agg, W1, b1.reshape(1, D), W2, b2.reshape(1, D))


# speedup vs baseline: 14.8524x; 1.0207x over previous
import functools

import jax
import jax.numpy as jnp
from jax import lax
from jax.experimental import pallas as pl
from jax.experimental.pallas import tpu as pltpu
from jax.experimental.pallas import tpu_sc as plsc

N, D, E = 10000, 128, 320000
NC, NS, L = 2, 16, 16
NW = NC * NS
STRIPE = 640
ACC_ROWS = NS * STRIPE
CH = 40
EW = E // NW
NCH = EW // CH
NB = 6
IDX_D = 5
G_D = 4
SC_LAG = NB - G_D
NSUP = (NCH + NB - 1) // NB
ESC = E // NC
KSC = ESC // 128
KPT = KSC // NS
SLAB = 4096


def _sc_aggregate(x, edge_index):
    mesh = plsc.VectorSubcoreMesh(core_axis_name="c", subcore_axis_name="s")

    @functools.partial(
        pl.kernel,
        out_type=[
            jax.ShapeDtypeStruct((NC, N, D), jnp.float32),
            jax.ShapeDtypeStruct((E,), jnp.int32),
            jax.ShapeDtypeStruct((E,), jnp.int32),
        ],
        mesh=mesh,
        scratch_types=[
            pltpu.VMEM_SHARED((ACC_ROWS, D), jnp.float32),
            [pltpu.VMEM((2, CH), jnp.int32) for _ in range(NB)],
            [pltpu.VMEM((CH,), jnp.int32) for _ in range(NB)],
            [pltpu.VMEM((CH, D), jnp.float32) for _ in range(NB)],
            pltpu.VMEM((2, SLAB), jnp.int32),
            pltpu.VMEM((16, D), jnp.float32),
            pltpu.SemaphoreType.DMA,
            pltpu.SemaphoreType.DMA,
            pltpu.SemaphoreType.DMA,
            pltpu.SemaphoreType.DMA,
        ],
    )
    def k(x_hbm, ei_hbm, out_hbm, row_hbm, col_hbm, acc, ei_r, dst_r,
          rows_r, tb, buf_v, sem_i, sem_g, sem_s, sem_z):
        c = lax.axis_index("c")
        s = lax.axis_index("s")
        ebase = (c * NS + s) * EW

        @pl.loop(0, 16)
        def _zb(i):
            @pl.loop(0, D, step=L)
            def _zl(j):
                buf_v[i, pl.ds(j, L)] = jnp.zeros((L,), jnp.float32)

        @pl.loop(0, STRIPE // 16)
        def _zs(i):
            pltpu.async_copy(buf_v, acc.at[pl.ds(s * STRIPE + i * 16, 16)],
                             sem_z)

        gbase = c * KSC + s * KPT + jnp.minimum(s, 2)

        def slab(off, sz):
            pltpu.sync_copy(ei_hbm.at[:, pl.ds(off, sz)],
                            tb.at[:, pl.ds(0, sz)])
            pltpu.sync_copy(tb.at[0, pl.ds(0, sz)],
                            row_hbm.at[pl.ds(off, sz)])
            pltpu.sync_copy(tb.at[1, pl.ds(0, sz)],
                            col_hbm.at[pl.ds(off, sz)])

        for j in range(KPT // 32):
            slab((gbase + j * 32) * 128, SLAB)
        toff = (gbase + (KPT // 32) * 32) * 128

        @pl.when(s < 2)
        def _t15(): slab(toff, (KPT % 32 + 1) * 128)

        @pl.when(s >= 2)
        def _t14(): slab(toff, (KPT % 32) * 128)

        @pl.loop(0, STRIPE // 16)
        def _zw(i):
            pltpu.make_async_copy(buf_v, acc.at[pl.ds(0, 16)], sem_z).wait()

        plsc.subcore_barrier()

        def issue_idx(g, b):
            pltpu.async_copy(row_hbm.at[pl.ds(ebase + g * CH, CH)],
                             ei_r[b].at[0], sem_i)
            pltpu.async_copy(col_hbm.at[pl.ds(ebase + g * CH, CH)],
                             ei_r[b].at[1], sem_i)

        def prep_gather(g, b):
            pltpu.make_async_copy(row_hbm.at[pl.ds(0, CH)], ei_r[0].at[0],
                                  sem_i).wait()
            pltpu.make_async_copy(row_hbm.at[pl.ds(0, CH)], ei_r[0].at[0],
                                  sem_i).wait()

            @pl.loop(0, CH, step=L)
            def _dst(i):
                r = ei_r[b][0, pl.ds(i, L)]
                cc = ei_r[b][1, pl.ds(i, L)]
                dst_r[b][pl.ds(i, L)] = jnp.where(r == cc, N, r)

            pltpu.async_copy(x_hbm.at[ei_r[b].at[1]], rows_r[b], sem_g)

        def retire(g, b):
            pltpu.make_async_copy(x_hbm.at[ei_r[b].at[1]], rows_r[b],
                                  sem_g).wait()
            pltpu.async_copy(rows_r[b], acc.at[dst_r[b]], sem_s, add=True)

        def drain_scatter():
            pltpu.make_async_copy(rows_r[0], acc.at[dst_r[0]], sem_s).wait()

        for p in range(IDX_D):
            issue_idx(p, p)
        for p in range(G_D):
            prep_gather(p, p)

        @pl.loop(0, NSUP)
        def _edges(sup):
            g0 = sup * NB
            for slot in range(NB):
                g = g0 + slot

                @pl.when(g + IDX_D < NCH)
                def _():
                    issue_idx(g + IDX_D, (slot + IDX_D) % NB)

                @pl.when((g >= SC_LAG) & (g - SC_LAG < NCH))
                def _():
                    drain_scatter()

                @pl.when(g + G_D < NCH)
                def _():
                    prep_gather(g + G_D, (slot + G_D) % NB)

                @pl.when(g < NCH)
                def _():
                    retire(g, slot)

        plsc.subcore_barrier()

        @pl.when(s < NS - 1)
        def _full():
            pltpu.sync_copy(acc.at[pl.ds(s * STRIPE, STRIPE)],
                            out_hbm.at[c, pl.ds(s * STRIPE, STRIPE)])

        @pl.when(s == NS - 1)
        def _tail():
            r0 = (NS - 1) * STRIPE
            pltpu.sync_copy(acc.at[pl.ds(r0, N - r0)],
                            out_hbm.at[c, pl.ds(r0, N - r0)])

    return k(x, edge_index)[0]


def _mlp(x, agg, W1, b1, W2, b2):
    BN = 1000

    def body(x_ref, a0_ref, a1_ref, W1_ref, b1_ref, W2_ref, b2_ref, o_ref):
        out = x_ref[...] + a0_ref[0] + a1_ref[0]
        h = lax.dot_general(out, W1_ref[...], (((1,), (0,)), ((), ())),
                            precision=lax.Precision.DEFAULT,
                            preferred_element_type=jnp.float32)
        h = jnp.maximum(h + b1_ref[...], 0.0)
        y = lax.dot_general(h, W2_ref[...], (((1,), (0,)), ((), ())),
                            precision=lax.Precision.DEFAULT,
                            preferred_element_type=jnp.float32)
        o_ref[...] = y + b2_ref[...]

    nd = pl.BlockSpec((BN, D), lambda i: (i, 0))
    a0 = pl.BlockSpec((1, BN, D), lambda i: (0, i, 0))
    a1 = pl.BlockSpec((1, BN, D), lambda i: (1, i, 0))
    w = pl.BlockSpec((D, D), lambda i: (0, 0))
    b = pl.BlockSpec((1, D), lambda i: (0, 0))
    return pl.pallas_call(
        body,
        grid=(N // BN,),
        in_specs=[nd, a0, a1, w, b, w, b],
        out_specs=nd,
        out_shape=jax.ShapeDtypeStruct((N, D), jnp.float32),
    )(x, agg, agg, W1, b1, W2, b2)


def kernel(x, edge_index, W1, b1, W2, b2):
    agg = _sc_aggregate(x, edge_index)
    return _mlp(x, agg, W1, b1.reshape(1, D), W2, b2.reshape(1, D))


# speedup vs baseline: 15.2107x; 1.0241x over previous
import functools

import jax
import jax.numpy as jnp
from jax import lax
from jax.experimental import pallas as pl
from jax.experimental.pallas import tpu as pltpu
from jax.experimental.pallas import tpu_sc as plsc

N, D, E = 10000, 128, 320000
NC, NS, L = 2, 16, 16
NW = NC * NS
STRIPE = 640
ACC_ROWS = NS * STRIPE
CH = 40
EW = E // NW
NCH = EW // CH
NB = 6
IDX_D = 5
G_D = 4
SC_LAG = NB - G_D
NSUP = (NCH + NB - 1) // NB
ESC = E // NC
KSC = ESC // 128
KPT = KSC // NS
SLAB = 2048


def _sc_aggregate(x, edge_index):
    mesh = plsc.VectorSubcoreMesh(core_axis_name="c", subcore_axis_name="s")

    @functools.partial(
        pl.kernel,
        out_type=[
            jax.ShapeDtypeStruct((NC, N, D), jnp.float32),
            jax.ShapeDtypeStruct((E,), jnp.int32),
            jax.ShapeDtypeStruct((E,), jnp.int32),
        ],
        mesh=mesh,
        scratch_types=[
            pltpu.VMEM_SHARED((ACC_ROWS, D), jnp.float32),
            [pltpu.VMEM((2, CH), jnp.int32) for _ in range(NB)],
            [pltpu.VMEM((CH,), jnp.int32) for _ in range(NB)],
            [pltpu.VMEM((CH, D), jnp.float32) for _ in range(NB)],
            [pltpu.VMEM((2, SLAB), jnp.int32) for _ in range(2)],
            pltpu.VMEM((16, D), jnp.float32),
            pltpu.SemaphoreType.DMA,
            pltpu.SemaphoreType.DMA,
            pltpu.SemaphoreType.DMA,
            pltpu.SemaphoreType.DMA,
            pltpu.SemaphoreType.DMA,
        ],
    )
    def k(x_hbm, ei_hbm, out_hbm, row_hbm, col_hbm, acc, ei_r, dst_r,
          rows_r, tb, buf_v, sem_i, sem_g, sem_s, sem_z, sem_p):
        c = lax.axis_index("c")
        s = lax.axis_index("s")
        ebase = (c * NS + s) * EW

        @pl.loop(0, 16)
        def _zb(i):
            @pl.loop(0, D, step=L)
            def _zl(j):
                buf_v[i, pl.ds(j, L)] = jnp.zeros((L,), jnp.float32)

        @pl.loop(0, STRIPE // 16)
        def _zs(i):
            pltpu.async_copy(buf_v, acc.at[pl.ds(s * STRIPE + i * 16, 16)],
                             sem_z)

        gbase = c * KSC + s * KPT + jnp.minimum(s, 2)
        NSLAB = KPT // (SLAB // 128)

        def slab_load(j, b, sz):
            pltpu.async_copy(ei_hbm.at[:, pl.ds((gbase + j * 16) * 128, sz)],
                             tb[b].at[:, pl.ds(0, sz)], sem_i)

        def slab_store(j, b, sz):
            pltpu.make_async_copy(ei_hbm.at[:, pl.ds(0, sz)],
                                  tb[b].at[:, pl.ds(0, sz)], sem_i).wait()
            off = (gbase + j * 16) * 128
            pltpu.async_copy(tb[b].at[0, pl.ds(0, sz)],
                             row_hbm.at[pl.ds(off, sz)], sem_p)
            pltpu.async_copy(tb[b].at[1, pl.ds(0, sz)],
                             col_hbm.at[pl.ds(off, sz)], sem_p)

        def slab_drain(sz):
            pltpu.make_async_copy(tb[0].at[0, pl.ds(0, sz)],
                                  row_hbm.at[pl.ds(0, sz)], sem_p).wait()
            pltpu.make_async_copy(tb[0].at[0, pl.ds(0, sz)],
                                  row_hbm.at[pl.ds(0, sz)], sem_p).wait()

        tail = (KPT % (SLAB // 128)) * 128
        for j in range(NSLAB):
            slab_load(j, j % 2, SLAB)
            if j >= 1:
                slab_store(j - 1, (j - 1) % 2, SLAB)
            if j >= 2:
                slab_drain(SLAB)
        slab_drain(SLAB)

        @pl.when(s < 2)
        def _t15():
            slab_load(NSLAB, 0, tail + 128)
        @pl.when(s >= 2)
        def _t14():
            slab_load(NSLAB, 0, tail)
        slab_store(NSLAB - 1, (NSLAB - 1) % 2, SLAB)
        slab_drain(SLAB)
        @pl.when(s < 2)
        def _t15b():
            slab_store(NSLAB, 0, tail + 128)
            slab_drain(tail + 128)
        @pl.when(s >= 2)
        def _t14b():
            slab_store(NSLAB, 0, tail)
            slab_drain(tail)

        @pl.loop(0, STRIPE // 16)
        def _zw(i):
            pltpu.make_async_copy(buf_v, acc.at[pl.ds(0, 16)], sem_z).wait()

        plsc.subcore_barrier()

        def issue_idx(g, b):
            pltpu.async_copy(row_hbm.at[pl.ds(ebase + g * CH, CH)],
                             ei_r[b].at[0], sem_i)
            pltpu.async_copy(col_hbm.at[pl.ds(ebase + g * CH, CH)],
                             ei_r[b].at[1], sem_i)

        def prep_gather(g, b):
            pltpu.make_async_copy(row_hbm.at[pl.ds(0, CH)], ei_r[0].at[0],
                                  sem_i).wait()
            pltpu.make_async_copy(row_hbm.at[pl.ds(0, CH)], ei_r[0].at[0],
                                  sem_i).wait()

            @pl.loop(0, CH, step=L)
            def _dst(i):
                r = ei_r[b][0, pl.ds(i, L)]
                cc = ei_r[b][1, pl.ds(i, L)]
                dst_r[b][pl.ds(i, L)] = jnp.where(r == cc, N, r)

            pltpu.async_copy(x_hbm.at[ei_r[b].at[1]], rows_r[b], sem_g)

        def retire(g, b):
            pltpu.make_async_copy(x_hbm.at[ei_r[b].at[1]], rows_r[b],
                                  sem_g).wait()
            pltpu.async_copy(rows_r[b], acc.at[dst_r[b]], sem_s, add=True)

        def drain_scatter():
            pltpu.make_async_copy(rows_r[0], acc.at[dst_r[0]], sem_s).wait()

        for p in range(IDX_D):
            issue_idx(p, p)
        for p in range(G_D):
            prep_gather(p, p)

        @pl.loop(0, NSUP)
        def _edges(sup):
            g0 = sup * NB
            for slot in range(NB):
                g = g0 + slot

                @pl.when(g + IDX_D < NCH)
                def _():
                    issue_idx(g + IDX_D, (slot + IDX_D) % NB)

                @pl.when((g >= SC_LAG) & (g - SC_LAG < NCH))
                def _():
                    drain_scatter()

                @pl.when(g + G_D < NCH)
                def _():
                    prep_gather(g + G_D, (slot + G_D) % NB)

                @pl.when(g < NCH)
                def _():
                    retire(g, slot)

        plsc.subcore_barrier()

        @pl.when(s < NS - 1)
        def _full():
            pltpu.sync_copy(acc.at[pl.ds(s * STRIPE, STRIPE)],
                            out_hbm.at[c, pl.ds(s * STRIPE, STRIPE)])

        @pl.when(s == NS - 1)
        def _tail():
            r0 = (NS - 1) * STRIPE
            pltpu.sync_copy(acc.at[pl.ds(r0, N - r0)],
                            out_hbm.at[c, pl.ds(r0, N - r0)])

    return k(x, edge_index)[0]


def _mlp(x, agg, W1, b1, W2, b2):
    BN = 2000

    def body(x_ref, a0_ref, a1_ref, W1_ref, b1_ref, W2_ref, b2_ref, o_ref):
        out = x_ref[...] + a0_ref[0] + a1_ref[0]
        h = lax.dot_general(out, W1_ref[...], (((1,), (0,)), ((), ())),
                            precision=lax.Precision.DEFAULT,
                            preferred_element_type=jnp.float32)
        h = jnp.maximum(h + b1_ref[...], 0.0)
        y = lax.dot_general(h, W2_ref[...], (((1,), (0,)), ((), ())),
                            precision=lax.Precision.DEFAULT,
                            preferred_element_type=jnp.float32)
        o_ref[...] = y + b2_ref[...]

    nd = pl.BlockSpec((BN, D), lambda i: (i, 0))
    a0 = pl.BlockSpec((1, BN, D), lambda i: (0, i, 0))
    a1 = pl.BlockSpec((1, BN, D), lambda i: (1, i, 0))
    w = pl.BlockSpec((D, D), lambda i: (0, 0))
    b = pl.BlockSpec((1, D), lambda i: (0, 0))
    return pl.pallas_call(
        body,
        grid=(N // BN,),
        in_specs=[nd, a0, a1, w, b, w, b],
        out_specs=nd,
        out_shape=jax.ShapeDtypeStruct((N, D), jnp.float32),
    )(x, agg, agg, W1, b1, W2, b2)


def kernel(x, edge_index, W1, b1, W2, b2):
    agg = _sc_aggregate(x, edge_index)
    return _mlp(x, agg, W1, b1.reshape(1, D), W2, b2.reshape(1, D))


# speedup vs baseline: 15.3868x; 1.0116x over previous
import functools

import jax
import jax.numpy as jnp
from jax import lax
from jax.experimental import pallas as pl
from jax.experimental.pallas import tpu as pltpu
from jax.experimental.pallas import tpu_sc as plsc

N, D, E = 10000, 128, 320000
NC, NS, L = 2, 16, 16
NW = NC * NS
STRIPE = 640
ACC_ROWS = NS * STRIPE
CH = 40
EW = E // NW
NCH = EW // CH
NB = 7
IDX_D = 6
G_D = 5
SC_LAG = NB - G_D
NSUP = (NCH + NB - 1) // NB
ESC = E // NC
KSC = ESC // 128
KPT = KSC // NS
SLAB = 2048


def _sc_aggregate(x, edge_index):
    mesh = plsc.VectorSubcoreMesh(core_axis_name="c", subcore_axis_name="s")

    @functools.partial(
        pl.kernel,
        out_type=[
            jax.ShapeDtypeStruct((NC, N, D), jnp.float32),
            jax.ShapeDtypeStruct((E,), jnp.int32),
            jax.ShapeDtypeStruct((E,), jnp.int32),
        ],
        mesh=mesh,
        scratch_types=[
            pltpu.VMEM_SHARED((ACC_ROWS, D), jnp.float32),
            [pltpu.VMEM((2, CH), jnp.int32) for _ in range(NB)],
            [pltpu.VMEM((CH,), jnp.int32) for _ in range(NB)],
            [pltpu.VMEM((CH, D), jnp.float32) for _ in range(NB)],
            [pltpu.VMEM((2, SLAB), jnp.int32) for _ in range(2)],
            pltpu.VMEM((16, D), jnp.float32),
            pltpu.SemaphoreType.DMA,
            pltpu.SemaphoreType.DMA,
            pltpu.SemaphoreType.DMA,
            pltpu.SemaphoreType.DMA,
            pltpu.SemaphoreType.DMA,
        ],
    )
    def k(x_hbm, ei_hbm, out_hbm, row_hbm, col_hbm, acc, ei_r, dst_r,
          rows_r, tb, buf_v, sem_i, sem_g, sem_s, sem_z, sem_p):
        c = lax.axis_index("c")
        s = lax.axis_index("s")
        ebase = (c * NS + s) * EW

        @pl.loop(0, 16)
        def _zb(i):
            @pl.loop(0, D, step=L)
            def _zl(j):
                buf_v[i, pl.ds(j, L)] = jnp.zeros((L,), jnp.float32)

        @pl.loop(0, STRIPE // 16)
        def _zs(i):
            pltpu.async_copy(buf_v, acc.at[pl.ds(s * STRIPE + i * 16, 16)],
                             sem_z)

        gbase = c * KSC + s * KPT + jnp.minimum(s, 2)
        NSLAB = KPT // (SLAB // 128)

        def slab_load(j, b, sz):
            pltpu.async_copy(ei_hbm.at[:, pl.ds((gbase + j * 16) * 128, sz)],
                             tb[b].at[:, pl.ds(0, sz)], sem_i)

        def slab_store(j, b, sz):
            pltpu.make_async_copy(ei_hbm.at[:, pl.ds(0, sz)],
                                  tb[b].at[:, pl.ds(0, sz)], sem_i).wait()
            off = (gbase + j * 16) * 128
            pltpu.async_copy(tb[b].at[0, pl.ds(0, sz)],
                             row_hbm.at[pl.ds(off, sz)], sem_p)
            pltpu.async_copy(tb[b].at[1, pl.ds(0, sz)],
                             col_hbm.at[pl.ds(off, sz)], sem_p)

        def slab_drain(sz):
            pltpu.make_async_copy(tb[0].at[0, pl.ds(0, sz)],
                                  row_hbm.at[pl.ds(0, sz)], sem_p).wait()
            pltpu.make_async_copy(tb[0].at[0, pl.ds(0, sz)],
                                  row_hbm.at[pl.ds(0, sz)], sem_p).wait()

        tail = (KPT % (SLAB // 128)) * 128
        for j in range(NSLAB):
            slab_load(j, j % 2, SLAB)
            if j >= 1:
                slab_store(j - 1, (j - 1) % 2, SLAB)
            if j >= 2:
                slab_drain(SLAB)
        slab_drain(SLAB)

        @pl.when(s < 2)
        def _t15():
            slab_load(NSLAB, 0, tail + 128)
        @pl.when(s >= 2)
        def _t14():
            slab_load(NSLAB, 0, tail)
        slab_store(NSLAB - 1, (NSLAB - 1) % 2, SLAB)
        slab_drain(SLAB)
        @pl.when(s < 2)
        def _t15b():
            slab_store(NSLAB, 0, tail + 128)
            slab_drain(tail + 128)
        @pl.when(s >= 2)
        def _t14b():
            slab_store(NSLAB, 0, tail)
            slab_drain(tail)

        @pl.loop(0, STRIPE // 16)
        def _zw(i):
            pltpu.make_async_copy(buf_v, acc.at[pl.ds(0, 16)], sem_z).wait()

        plsc.subcore_barrier()

        def issue_idx(g, b):
            pltpu.async_copy(row_hbm.at[pl.ds(ebase + g * CH, CH)],
                             ei_r[b].at[0], sem_i)
            pltpu.async_copy(col_hbm.at[pl.ds(ebase + g * CH, CH)],
                             ei_r[b].at[1], sem_i)

        def prep_gather(g, b):
            pltpu.make_async_copy(row_hbm.at[pl.ds(0, CH)], ei_r[0].at[0],
                                  sem_i).wait()
            pltpu.make_async_copy(row_hbm.at[pl.ds(0, CH)], ei_r[0].at[0],
                                  sem_i).wait()

            @pl.loop(0, CH, step=L)
            def _dst(i):
                r = ei_r[b][0, pl.ds(i, L)]
                cc = ei_r[b][1, pl.ds(i, L)]
                dst_r[b][pl.ds(i, L)] = jnp.where(r == cc, N, r)

            pltpu.async_copy(x_hbm.at[ei_r[b].at[1]], rows_r[b], sem_g)

        def retire(g, b):
            pltpu.make_async_copy(x_hbm.at[ei_r[b].at[1]], rows_r[b],
                                  sem_g).wait()
            pltpu.async_copy(rows_r[b], acc.at[dst_r[b]], sem_s, add=True)

        def drain_scatter():
            pltpu.make_async_copy(rows_r[0], acc.at[dst_r[0]], sem_s).wait()

        for p in range(IDX_D):
            issue_idx(p, p)
        for p in range(G_D):
            prep_gather(p, p)

        @pl.loop(0, NSUP)
        def _edges(sup):
            g0 = sup * NB
            for slot in range(NB):
                g = g0 + slot

                @pl.when(g + IDX_D < NCH)
                def _():
                    issue_idx(g + IDX_D, (slot + IDX_D) % NB)

                @pl.when((g >= SC_LAG) & (g - SC_LAG < NCH))
                def _():
                    drain_scatter()

                @pl.when(g + G_D < NCH)
                def _():
                    prep_gather(g + G_D, (slot + G_D) % NB)

                @pl.when(g < NCH)
                def _():
                    retire(g, slot)

        plsc.subcore_barrier()

        @pl.when(s < NS - 1)
        def _full():
            pltpu.sync_copy(acc.at[pl.ds(s * STRIPE, STRIPE)],
                            out_hbm.at[c, pl.ds(s * STRIPE, STRIPE)])

        @pl.when(s == NS - 1)
        def _tail():
            r0 = (NS - 1) * STRIPE
            pltpu.sync_copy(acc.at[pl.ds(r0, N - r0)],
                            out_hbm.at[c, pl.ds(r0, N - r0)])

    return k(x, edge_index)[0]


def _mlp(x, agg, W1, b1, W2, b2):
    BN = 2000

    def body(x_ref, a0_ref, a1_ref, W1_ref, b1_ref, W2_ref, b2_ref, o_ref):
        out = x_ref[...] + a0_ref[0] + a1_ref[0]
        h = lax.dot_general(out, W1_ref[...], (((1,), (0,)), ((), ())),
                            precision=lax.Precision.DEFAULT,
                            preferred_element_type=jnp.float32)
        h = jnp.maximum(h + b1_ref[...], 0.0)
        y = lax.dot_general(h, W2_ref[...], (((1,), (0,)), ((), ())),
                            precision=lax.Precision.DEFAULT,
                            preferred_element_type=jnp.float32)
        o_ref[...] = y + b2_ref[...]

    nd = pl.BlockSpec((BN, D), lambda i: (i, 0))
    a0 = pl.BlockSpec((1, BN, D), lambda i: (0, i, 0))
    a1 = pl.BlockSpec((1, BN, D), lambda i: (1, i, 0))
    w = pl.BlockSpec((D, D), lambda i: (0, 0))
    b = pl.BlockSpec((1, D), lambda i: (0, 0))
    return pl.pallas_call(
        body,
        grid=(N // BN,),
        in_specs=[nd, a0, a1, w, b, w, b],
        out_specs=nd,
        out_shape=jax.ShapeDtypeStruct((N, D), jnp.float32),
    )(x, agg, agg, W1, b1, W2, b2)


def kernel(x, edge_index, W1, b1, W2, b2):
    agg = _sc_aggregate(x, edge_index)
    return _mlp(x, agg, W1, b1.reshape(1, D), W2, b2.reshape(1, D))
